# baseline (speedup 1.0000x reference)
import jax
import jax.numpy as jnp
import numpy as np
from jax import lax
from jax.experimental import pallas as pl
from jax.experimental.pallas import tpu as pltpu
from jax.experimental.pallas import tpu_sc as plsc

N_NODES = 10000
N_EDGES = 320000
D = 128
NR = 16

NC = 2
NS = 16
NW = NC * NS
EPW = N_EDGES // NW
CHUNK = 80
N_CHUNKS = EPW // CHUNK
N_PAIRS = (N_CHUNKS - 1) // 2
EB = 4
LANES = 16

ACC_BLOCK = 8000

_PERM = np.concatenate(
    [np.arange(32 * blk, 32 * (blk + 1)).reshape(16, 2).T.reshape(32)
     for blk in range(D // 32)])


def _tc1_body(x_ref, wji_ref, b_ref, idxj_ref, idxi_ref, t_ref, idx_ref):
    x = x_ref[...]
    t_ref[0:N_NODES, :] = jnp.dot(x, wji_ref[0:D, :],
                                  preferred_element_type=jnp.float32)
    t_ref[N_NODES:2 * N_NODES, :] = (
        jnp.dot(x, wji_ref[D:2 * D, :], preferred_element_type=jnp.float32)
        + b_ref[...]
    )
    idx_ref[0] = idxj_ref[...]
    idx_ref[1] = idxi_ref[...] + N_NODES


def _tc1(x, wji_perm, b_perm, idx_j, idx_i):
    n_idx_rows = N_EDGES // D
    return pl.pallas_call(
        _tc1_body,
        out_shape=[
            jax.ShapeDtypeStruct((2 * N_NODES, D), jnp.float32),
            jax.ShapeDtypeStruct((2, n_idx_rows, D), jnp.int32),
        ],
    )(x, wji_perm, b_perm.reshape(1, D),
      idx_j.reshape(n_idx_rows, D), idx_i.reshape(n_idx_rows, D))


def _sc_body(t_hbm, idx_hbm, gsum_hbm,
             idx_v0, idx_v1, rows_j, rows_i, out_v,
             sem_g0, sem_g1, sem_o0, sem_o1):
    wid = lax.axis_index("s") * NC + lax.axis_index("c")
    base_w = wid * EPW
    sem_g = (sem_g0, sem_g1)
    sem_o = (sem_o0, sem_o1)
    rows = ((rows_j.at[0], rows_i.at[0]), (rows_j.at[1], rows_i.at[1]))
    outb = (out_v.at[0], out_v.at[1])

    pltpu.sync_copy(idx_hbm.at[0, wid, 0, :], idx_v0)
    pltpu.sync_copy(idx_hbm.at[1, wid, 0, :], idx_v1)

    def issue_in(c, b):
        off = c * CHUNK
        pltpu.async_copy(t_hbm.at[idx_v0.at[pl.ds(off, CHUNK)]],
                         rows[b][0], sem_g[b])
        pltpu.async_copy(t_hbm.at[idx_v1.at[pl.ds(off, CHUNK)]],
                         rows[b][1], sem_g[b])

    def wait_in(b):
        pltpu.make_async_copy(t_hbm.at[idx_v0.at[pl.ds(0, CHUNK)]],
                              rows[b][0], sem_g[b]).wait()
        pltpu.make_async_copy(t_hbm.at[idx_v1.at[pl.ds(0, CHUNK)]],
                              rows[b][1], sem_g[b]).wait()

    def wait_out(b):
        pltpu.make_async_copy(outb[b], gsum_hbm.at[pl.ds(0, CHUNK)],
                              sem_o[b]).wait()

    def compute_store(c, b):
        rj, ri = rows[b]
        ov = outb[b]

        def eb_body(i, _):
            e0 = i * EB
            for ep in range(EB):
                e = e0 + ep
                for g in range(D // 32):
                    s0l = pl.ds(32 * g, LANES)
                    s1l = pl.ds(32 * g + LANES, LANES)
                    s0 = rj[e, s0l] + ri[e, s0l]
                    s1 = rj[e, s1l] + ri[e, s1l]
                    ov[e, pl.ds(LANES * g, LANES)] = plsc.bitcast(
                        plsc.pack(s0, s1, format=plsc.PackFormat.INTERLEAVED),
                        jnp.int32)
            return 0

        lax.fori_loop(0, CHUNK // EB, eb_body, 0)
        pltpu.async_copy(ov, gsum_hbm.at[pl.ds(base_w + c * CHUNK, CHUNK)],
                         sem_o[b])

    issue_in(0, 0)
    issue_in(1, 1)

    def pair_body(p, _):
        c0 = 2 * p
        for b in (0, 1):
            c = c0 + b
            wait_in(b)

            @pl.when(c >= 2)
            def _():
                wait_out(b)

            compute_store(c, b)

            @pl.when(c + 2 < N_CHUNKS)
            def _():
                issue_in(c + 2, b)

        return 0

    lax.fori_loop(0, N_PAIRS, pair_body, 0)
    wait_in(0)
    wait_out(0)
    compute_store(N_CHUNKS - 1, 0)
    wait_out(1)
    wait_out(0)


def _sc_gsum(t, idx_cat):
    mesh = plsc.VectorSubcoreMesh(core_axis_name="c", subcore_axis_name="s")
    return pl.kernel(
        _sc_body,
        out_type=jax.ShapeDtypeStruct((N_EDGES, D // 2), jnp.int32),
        mesh=mesh,
        compiler_params=pltpu.CompilerParams(needs_layout_passes=False),
        scratch_types=[
            pltpu.VMEM((EPW,), jnp.int32),
            pltpu.VMEM((EPW,), jnp.int32),
            pltpu.VMEM((2, CHUNK, D), jnp.float32),
            pltpu.VMEM((2, CHUNK, D), jnp.float32),
            pltpu.VMEM((2, CHUNK, D // 2), jnp.int32),
        ] + [pltpu.SemaphoreType.DMA] * 4,
    )(t, idx_cat)


def _tc2_body(gsum_ref, rbf_ref, wrbf_ref, wedge_ref, out_ref):
    wr = jnp.dot(wrbf_ref[...], wedge_ref[2 * D:3 * D, :],
                 preferred_element_type=jnp.float32)
    acc = jnp.dot(rbf_ref[...], wr, preferred_element_type=jnp.float32)
    t = gsum_ref[...].astype(jnp.float32) + acc
    out_ref[...] = t / (1.0 + jnp.exp(-t))


def _tc2(gsum, rbf, W_rbf, W_edge):
    full = lambda shape: pl.BlockSpec(shape, lambda i: tuple(0 for _ in shape))
    return pl.pallas_call(
        _tc2_body,
        grid=(N_EDGES // ACC_BLOCK,),
        in_specs=[
            pl.BlockSpec((ACC_BLOCK, D), lambda i: (i, 0)),
            pl.BlockSpec((ACC_BLOCK, NR), lambda i: (i, 0)),
            full((NR, D)),
            full((3 * D, D)),
        ],
        out_specs=pl.BlockSpec((ACC_BLOCK, D), lambda i: (i, 0)),
        out_shape=jax.ShapeDtypeStruct((N_EDGES, D), jnp.float32),
    )(gsum, rbf, W_rbf, W_edge)


def kernel(x, rbf, idx_i, idx_j, W_rbf, W_edge, b_edge):
    idx_i = idx_i.astype(jnp.int32)
    idx_j = idx_j.astype(jnp.int32)
    wji_perm = W_edge[0:2 * D][:, _PERM]
    b_perm = b_edge[_PERM]
    t, idx_cat = _tc1(x, wji_perm, b_perm, idx_j, idx_i)
    gsum_i32 = _sc_gsum(t, idx_cat.reshape(2, NW, 1, EPW))
    gsum = lax.bitcast_convert_type(gsum_i32, jnp.bfloat16).reshape(N_EDGES, D)
    return _tc2(gsum, rbf, W_rbf, W_edge)
